# Initial kernel scaffold; baseline (speedup 1.0000x reference)
import jax
import jax.numpy as jnp
from jax.experimental import pallas as pl


def kernel(bnd_nodes, bnd_edge_index, bnd_edge_attr, W, att_src, att_dst, We, att_edge, bias, ln_g, ln_b):
    raise NotImplementedError("write your pallas kernel here")


# speedup vs baseline: 12.9572x; 12.9572x over previous
import functools

import jax
import jax.numpy as jnp
from jax import lax
from jax.experimental import pallas as pl
from jax.experimental.pallas import tpu as pltpu
from jax.experimental.pallas import tpu_sc as plsc

N = 10000
E = 320000
D = 128
ED = 16
DEPTH = 4

NC = 2
NS = 16
NW = NC * NS
EPT = E // NW
NB = EPT // 16
NROWPAD = 640 * 16


_BLK = 1000
_GRID = N // _BLK


def _edge_att_body(ea_ref, we_ref, ae_ref, out_ref):
    for l in range(DEPTH):
        v = jnp.sum(we_ref[l] * ae_ref[l][None, :], axis=1)
        m = jnp.dot(ea_ref[...], v.reshape(ED, 1),
                    preferred_element_type=jnp.float32)
        out_ref[:, pl.ds(l, 1)] = m


def _edge_att(ea, We, att_edge):
    eb = 4000
    return pl.pallas_call(
        _edge_att_body,
        grid=(E // eb,),
        in_specs=[
            pl.BlockSpec((eb, ED), lambda i: (i, 0)),
            pl.BlockSpec((DEPTH, ED, D), lambda i: (0, 0, 0)),
            pl.BlockSpec((DEPTH, D), lambda i: (0, 0)),
        ],
        out_specs=pl.BlockSpec((eb, DEPTH), lambda i: (i, 0)),
        out_shape=jax.ShapeDtypeStruct((E, DEPTH), jnp.float32),
    )(ea, We, att_edge)


def _dense0_body(x_ref, w_ref, as_ref, ad_ref, h_ref, ss_ref, sd_ref):
    h = jnp.dot(x_ref[...], w_ref[...], preferred_element_type=jnp.float32)
    h_ref[...] = h
    ss_ref[...] = jnp.dot(h, as_ref[...], preferred_element_type=jnp.float32)
    sd_ref[...] = jnp.dot(h, ad_ref[...], preferred_element_type=jnp.float32)


def _dense0(x, W0, a_s, a_d):
    return pl.pallas_call(
        _dense0_body,
        grid=(_GRID,),
        in_specs=[
            pl.BlockSpec((_BLK, D), lambda i: (i, 0)),
            pl.BlockSpec((D, D), lambda i: (0, 0)),
            pl.BlockSpec((D, 1), lambda i: (0, 0)),
            pl.BlockSpec((D, 1), lambda i: (0, 0)),
        ],
        out_specs=[
            pl.BlockSpec((_BLK, D), lambda i: (i, 0)),
            pl.BlockSpec((_BLK, 1), lambda i: (i, 0)),
            pl.BlockSpec((_BLK, 1), lambda i: (i, 0)),
        ],
        out_shape=[
            jax.ShapeDtypeStruct((N, D), jnp.float32),
            jax.ShapeDtypeStruct((N, 1), jnp.float32),
            jax.ShapeDtypeStruct((N, 1), jnp.float32),
        ],
    )(x, W0, a_s, a_d)


def _densei_body(u_ref, s_ref, b_ref, w_ref, as_ref, ad_ref,
                 h_ref, ss_ref, sd_ref):
    u = u_ref[0] + u_ref[1]
    s = s_ref[0] + s_ref[1]
    x = jnp.maximum(u / (s + 1e-16) + b_ref[...], 0.0)
    h = jnp.dot(x, w_ref[...], preferred_element_type=jnp.float32)
    h_ref[...] = h
    ss_ref[...] = jnp.dot(h, as_ref[...], preferred_element_type=jnp.float32)
    sd_ref[...] = jnp.dot(h, ad_ref[...], preferred_element_type=jnp.float32)


def _densei(u, s, b, Wi, a_s, a_d):
    return pl.pallas_call(
        _densei_body,
        grid=(_GRID,),
        in_specs=[
            pl.BlockSpec((NC, _BLK, D), lambda i: (0, i, 0)),
            pl.BlockSpec((NC, _BLK, 1), lambda i: (0, i, 0)),
            pl.BlockSpec((1, D), lambda i: (0, 0)),
            pl.BlockSpec((D, D), lambda i: (0, 0)),
            pl.BlockSpec((D, 1), lambda i: (0, 0)),
            pl.BlockSpec((D, 1), lambda i: (0, 0)),
        ],
        out_specs=[
            pl.BlockSpec((_BLK, D), lambda i: (i, 0)),
            pl.BlockSpec((_BLK, 1), lambda i: (i, 0)),
            pl.BlockSpec((_BLK, 1), lambda i: (i, 0)),
        ],
        out_shape=[
            jax.ShapeDtypeStruct((N, D), jnp.float32),
            jax.ShapeDtypeStruct((N, 1), jnp.float32),
            jax.ShapeDtypeStruct((N, 1), jnp.float32),
        ],
    )(u, s, b, Wi, a_s, a_d)


def _final_body(u_ref, s_ref, b_ref, g_ref, lb_ref, out_ref, acc_ref):
    step = pl.program_id(0)

    @pl.when(step == 0)
    def _():
        acc_ref[...] = jnp.zeros_like(acc_ref)

    u = u_ref[0] + u_ref[1]
    s = s_ref[0] + s_ref[1]
    x = u / (s + 1e-16) + b_ref[...]
    acc_ref[...] += jnp.sum(x, axis=0, keepdims=True)

    @pl.when(step == _GRID - 1)
    def _():
        pooled = acc_ref[...] * (1.0 / N)
        mu = jnp.mean(pooled, axis=-1, keepdims=True)
        var = jnp.mean((pooled - mu) ** 2, axis=-1, keepdims=True)
        out_ref[...] = ((pooled - mu) * lax.rsqrt(var + 1e-5) * g_ref[...]
                        + lb_ref[...])


def _final(u, s, b, g, lb):
    return pl.pallas_call(
        _final_body,
        grid=(_GRID,),
        in_specs=[
            pl.BlockSpec((NC, _BLK, D), lambda i: (0, i, 0)),
            pl.BlockSpec((NC, _BLK, 1), lambda i: (0, i, 0)),
            pl.BlockSpec((1, D), lambda i: (0, 0)),
            pl.BlockSpec((1, D), lambda i: (0, 0)),
            pl.BlockSpec((1, D), lambda i: (0, 0)),
        ],
        out_specs=pl.BlockSpec((1, D), lambda i: (0, 0)),
        out_shape=jax.ShapeDtypeStruct((1, D), jnp.float32),
        scratch_shapes=[pltpu.VMEM((1, D), jnp.float32)],
    )(u, s, b, g, lb)


_CHUNK = 125
_NCHUNK = NB // _CHUNK
_RPT = N // NS


def _sc_body(src_hbm, dst_hbm, ae_hbm, ssrc_hbm, sdst_hbm, h_hbm,
             u_hbm, s_hbm,
             src_c, dst_c, ae_c, ssrc_tbl, sdst_tbl, s_part,
             hb, sb, szb, idxm, u_sh, s_sh):
    cid = lax.axis_index("c")
    sid = lax.axis_index("s")
    wid = cid * NS + sid

    pltpu.sync_copy(ssrc_hbm, ssrc_tbl)
    pltpu.sync_copy(sdst_hbm, sdst_tbl)

    z16 = jnp.zeros((16,), jnp.float32)
    for r in range(16):
        for c in range(8):
            sb[r, pl.ds(c * 16, 16)] = z16
    for r in range(40):
        szb[r] = z16
    iota16 = lax.iota(jnp.int32, 16)
    for r in range(40):
        idxm[r] = iota16 + r * 16

    def zp(r, carry):
        s_part[r] = z16
        return carry
    lax.fori_loop(0, 640, zp, 0)

    ubase = sid * _RPT

    def zu(m, carry):
        pltpu.sync_copy(sb, u_sh.at[pl.ds(ubase + m * 16, 16)])
        return carry
    lax.fori_loop(0, _RPT // 16, zu, 0)
    pltpu.sync_copy(sb.at[pl.ds(0, _RPT % 16)],
                    u_sh.at[pl.ds(ubase + (_RPT // 16) * 16, _RPT % 16)])
    pltpu.sync_copy(szb, s_sh.at[pl.ds(sid * 40, 40)])

    plsc.subcore_barrier()

    def batch(i, carry):
        srcv = src_c[i]
        dstv = dst_c[i]
        aev = ae_c[i]
        gs = plsc.load_gather(ssrc_tbl, [srcv])
        gd = plsc.load_gather(sdst_tbl, [dstv])
        al = gs + gd + aev
        al = jnp.where(al >= 0.0, al, al * 0.2)
        ex = jnp.exp(al)
        plsc.addupdate_scatter(
            s_part,
            [lax.shift_right_logical(dstv, 4), lax.bitwise_and(dstv, 15)],
            ex)
        pltpu.sync_copy(h_hbm.at[srcv], hb)
        for j in range(16):
            exj = lax.gather(
                ex, jnp.full((16, 1), j, jnp.int32),
                lax.GatherDimensionNumbers(offset_dims=(),
                                           collapsed_slice_dims=(0,),
                                           start_index_map=(0,)),
                (1,), mode=lax.GatherScatterMode.PROMISE_IN_BOUNDS)
            for c in range(8):
                sl = pl.ds(c * 16, 16)
                sb[j, sl] = hb[j, sl] * exj
        pltpu.sync_copy(sb, u_sh.at[dst_c.at[i]], add=True)
        return carry

    for ch in range(_NCHUNK):
        csl = pl.ds(ch * _CHUNK, _CHUNK)
        pltpu.sync_copy(src_hbm.at[wid, csl], src_c)
        pltpu.sync_copy(dst_hbm.at[wid, csl], dst_c)
        pltpu.sync_copy(ae_hbm.at[wid, csl], ae_c)
        lax.fori_loop(0, _CHUNK, batch, 0)

    for m in range(40):
        pltpu.sync_copy(s_part.at[pl.ds(m * 16, 16)],
                        s_sh.at[idxm.at[m]], add=True)

    plsc.subcore_barrier()

    pltpu.sync_copy(u_sh.at[pl.ds(ubase, _RPT)],
                    u_hbm.at[cid, pl.ds(ubase, _RPT)])

    @pl.when(sid == 0)
    def _():
        pltpu.sync_copy(s_sh, s_hbm.at[cid])


@functools.partial(
    pl.kernel,
    out_type=[
        jax.ShapeDtypeStruct((NC, N, D), jnp.float32),
        jax.ShapeDtypeStruct((NC, 640, 16), jnp.float32),
    ],
    mesh=plsc.VectorSubcoreMesh(core_axis_name="c", subcore_axis_name="s"),
    compiler_params=pltpu.CompilerParams(needs_layout_passes=False,
                                         use_tc_tiling_on_sc=False),
    scratch_types=[
        pltpu.VMEM((_CHUNK, 16), jnp.int32),
        pltpu.VMEM((_CHUNK, 16), jnp.int32),
        pltpu.VMEM((_CHUNK, 16), jnp.float32),
        pltpu.VMEM((N,), jnp.float32),
        pltpu.VMEM((N,), jnp.float32),
        pltpu.VMEM((640, 16), jnp.float32),
        pltpu.VMEM((16, D), jnp.float32),
        pltpu.VMEM((16, D), jnp.float32),
        pltpu.VMEM((40, 16), jnp.float32),
        pltpu.VMEM((40, 16), jnp.int32),
        pltpu.VMEM_SHARED((N, D), jnp.float32),
        pltpu.VMEM_SHARED((640, 16), jnp.float32),
    ],
)
def _sc_layer(src3, dst3, ae3, ssrc, sdst, h, u_out, s_out, *scratch):
    _sc_body(src3, dst3, ae3, ssrc, sdst, h, u_out, s_out, *scratch)


def kernel(bnd_nodes, bnd_edge_index, bnd_edge_attr, W, att_src, att_dst,
           We, att_edge, bias, ln_g, ln_b):
    src3 = bnd_edge_index[0].reshape(NW, NB, 16)
    dst3 = bnd_edge_index[1].reshape(NW, NB, 16)
    alphaE = _edge_att(bnd_edge_attr, We, att_edge)
    ae3 = alphaE.T.reshape(DEPTH, NW, NB, 16)

    u = s = None
    for i in range(DEPTH):
        a_s = att_src[i].reshape(D, 1)
        a_d = att_dst[i].reshape(D, 1)
        if i == 0:
            h, ssrc, sdst = _dense0(bnd_nodes, W[0], a_s, a_d)
        else:
            h, ssrc, sdst = _densei(u, s, bias[i - 1].reshape(1, D),
                                    W[i], a_s, a_d)
        u, s_raw = _sc_layer(src3, dst3, ae3[i],
                             ssrc.reshape(N), sdst.reshape(N), h)
        s = s_raw.reshape(NC, NROWPAD, 1)

    return _final(u, s, bias[DEPTH - 1].reshape(1, D),
                  ln_g.reshape(1, D), ln_b.reshape(1, D))
